# baseline (speedup 1.0000x reference)
import jax
import jax.numpy as jnp
from jax.experimental import pallas as pl
from jax.experimental.pallas import tpu as pltpu

NBATCH = 64
NPRIORS = 8732
NCLASSES = 21
NOBJ = 20
OVERLAP_THRESHOLD = 0.5
NEGPOS_RATIO = 3
ALPHA = 1.0

RS = 72
LS = 128
PPAD = RS * LS

SEARCH_WAYS = 8
SEARCH_STEPS = 8
IMGS = 1


class _Fold:
    def __init__(self, op):
        self.op = op
        self.stack = []

    def push(self, v):
        r = 0
        while self.stack and self.stack[-1][0] == r:
            pr, pv = self.stack.pop()
            v = self.op(pv, v)
            r = pr + 1
        self.stack.append((r, v))

    def result(self):
        acc = self.stack[0][1]
        for _, v in self.stack[1:]:
            acc = self.op(acc, v)
        return acc


def _mbl_kernel(locs_ref, confs_ref, tgt_ref, priors_ref, out_ref):
    for i in range(IMGS):
        _one_image(locs_ref, confs_ref, tgt_ref, priors_ref, out_ref, i)


def _one_image(locs_ref, confs_ref, tgt_ref, priors_ref, out_ref, img):
    f32 = jnp.float32
    row_iota = jax.lax.broadcasted_iota(jnp.int32, (RS, LS), 0)
    lane_iota = jax.lax.broadcasted_iota(jnp.int32, (RS, LS), 1)
    idx = row_iota * LS + lane_iota
    valid = idx < NPRIORS

    px1 = priors_ref[0]
    py1 = priors_ref[1]
    px2 = priors_ref[2]
    py2 = priors_ref[3]
    area_b = priors_ref[4]
    pcx = priors_ref[5]
    pcy = priors_ref[6]
    rw10 = priors_ref[7]
    rh10 = priors_ref[8]
    rw = priors_ref[9]
    rh = priors_ref[10]

    def argmax_combine(a, b):
        va, ja = a
        vb, jb = b
        take = vb > va
        return jnp.maximum(va, vb), jnp.where(take, jb, ja)

    bps = []
    am = _Fold(argmax_combine)
    for j in range(NOBJ):
        ax1 = tgt_ref[img, j, 0]
        ay1 = tgt_ref[img, j, 1]
        ax2 = tgt_ref[img, j, 2]
        ay2 = tgt_ref[img, j, 3]
        area_a = (ax2 - ax1) * (ay2 - ay1)
        wx = jnp.maximum(jnp.minimum(ax2, px2) - jnp.maximum(ax1, px1), 0.0)
        wy = jnp.maximum(jnp.minimum(ay2, py2) - jnp.maximum(ay1, py1), 0.0)
        inter = wx * wy
        ov = inter / (area_a + area_b - inter)
        cmax = jnp.max(ov, axis=0, keepdims=True)
        rmin = jnp.min(jnp.where(ov == cmax, idx, PPAD),
                       axis=0, keepdims=True)
        rm = jnp.max(cmax, axis=1, keepdims=True)
        bps.append(jnp.min(jnp.where(cmax == rm, rmin, PPAD),
                           axis=1, keepdims=True))
        am.push((ov, jnp.full((RS, LS), f32(j))))
    best_ov, best_idx = am.result()

    wf = _Fold(jnp.maximum)
    for j in range(NOBJ):
        wf.push(jnp.where(idx == bps[j], f32(j), -1.0))
    winner = wf.result()
    forced = winner >= 0
    best_idx = jnp.where(forced, winner, best_idx)
    best_ov = jnp.where(forced, 1.0, best_ov)

    lab = jnp.zeros((RS, LS), f32)
    mx1 = jnp.zeros((RS, LS), f32)
    my1 = jnp.zeros((RS, LS), f32)
    mx2 = jnp.zeros((RS, LS), f32)
    my2 = jnp.zeros((RS, LS), f32)
    for j in range(NOBJ):
        sel = best_idx == f32(j)
        lab = lab + jnp.where(sel, tgt_ref[img, j, 4], 0.0)
        mx1 = mx1 + jnp.where(sel, tgt_ref[img, j, 0], 0.0)
        my1 = my1 + jnp.where(sel, tgt_ref[img, j, 1], 0.0)
        mx2 = mx2 + jnp.where(sel, tgt_ref[img, j, 2], 0.0)
        my2 = my2 + jnp.where(sel, tgt_ref[img, j, 3], 0.0)

    pos = jnp.logical_and(best_ov >= OVERLAP_THRESHOLD, valid)
    lab = jnp.where(pos, lab, 0.0)
    n_pos = jnp.sum(jnp.where(pos, 1.0, 0.0), keepdims=True)

    cx = (mx1 + mx2) / 2
    cy = (my1 + my2) / 2
    w = mx2 - mx1
    h = my2 - my1
    g_cx = (cx - pcx) * rw10
    g_cy = (cy - pcy) * rh10
    g_w = jnp.log(w * rw + 1e-10) * 5
    g_h = jnp.log(h * rh + 1e-10) * 5

    loc_abs = (jnp.where(pos, jnp.abs(locs_ref[img, 0] - g_cx), 0.0)
               + jnp.where(pos, jnp.abs(locs_ref[img, 1] - g_cy), 0.0)
               + jnp.where(pos, jnp.abs(locs_ref[img, 2] - g_w), 0.0)
               + jnp.where(pos, jnp.abs(locs_ref[img, 3] - g_h), 0.0))
    loc_sum = jnp.sum(loc_abs, keepdims=True)

    sf = _Fold(jnp.add)
    for c in range(NCLASSES):
        sf.push(jnp.exp(confs_ref[img, c]))
    lse = jnp.log(sf.result())
    pf = _Fold(jnp.add)
    pf.push(jnp.where(lab == 0.0, confs_ref[img, 0], 0.0))
    for c in range(1, NCLASSES):
        pf.push(jnp.where(lab == f32(c), confs_ref[img, c], 0.0))
    ce = lse - pf.result()

    pos_ce_sum = jnp.sum(jnp.where(pos, ce, 0.0), keepdims=True)

    ce_neg = jnp.where(jnp.logical_or(pos, jnp.logical_not(valid)), 0.0, ce)
    k = jnp.minimum(n_pos * NEGPOS_RATIO, f32(NPRIORS))

    lo = jnp.full((1, 1), -1.0, f32)
    wdt = jnp.max(ce_neg, keepdims=True) + 2.0
    inv = 1.0 / SEARCH_WAYS
    for _ in range(SEARCH_STEPS):
        ws = wdt * inv
        cnts = [jnp.sum(jnp.where(ce_neg > lo + ws * i, 1.0, 0.0),
                        keepdims=True) for i in range(1, SEARCH_WAYS)]
        jf = _Fold(jnp.add)
        for c in cnts:
            jf.push(jnp.where(c > k, 1.0, 0.0))
        lo = lo + ws * jf.result()
        wdt = ws
    hi = lo + wdt
    above = ce_neg > hi
    cnt_hi = jnp.sum(jnp.where(above, 1.0, 0.0), keepdims=True)
    top_sum = jnp.sum(jnp.where(above, ce_neg, 0.0), keepdims=True)
    vnext = jnp.max(jnp.where(above, -1.0, ce_neg), keepdims=True)
    min_above = jnp.min(jnp.where(above, ce_neg, 3.4e38), keepdims=True)
    rem = k - cnt_hi
    hard_sum = top_sum + jnp.where(rem >= 0, rem * vnext, rem * min_above)

    lane8 = jax.lax.broadcasted_iota(jnp.int32, (1, 8), 1)
    row = (jnp.where(lane8 == 0, n_pos, 0.0)
           + jnp.where(lane8 == 1, loc_sum, 0.0)
           + jnp.where(lane8 == 2, pos_ce_sum, 0.0)
           + jnp.where(lane8 == 3, hard_sum, 0.0))
    out_ref[img] = row


NSPLIT = 2


def kernel(pred_locs, pred_confs, targets, priors):
    pad = PPAD - NPRIORS
    hb = NBATCH // NSPLIT

    pcx, pcy, pw, ph = priors[:, 0], priors[:, 1], priors[:, 2], priors[:, 3]
    px1 = pcx - pw / 2
    py1 = pcy - ph / 2
    px2 = pcx + pw / 2
    py2 = pcy + ph / 2
    area_b = (px2 - px1) * (py2 - py1)
    chans = jnp.stack([px1, py1, px2, py2, area_b, pcx, pcy,
                       10.0 / pw, 10.0 / ph, 1.0 / pw, 1.0 / ph,
                       jnp.zeros_like(pw)])
    priors_p = jnp.pad(chans, ((0, 0), (0, pad))).reshape(12, RS, LS)

    outs = []
    for h in range(NSPLIT):
        sl = slice(h * hb, (h + 1) * hb)
        locs_p = jnp.pad(jnp.transpose(pred_locs[sl], (0, 2, 1)),
                         ((0, 0), (0, 0), (0, pad))).reshape(hb, 4, RS, LS)
        confs_p = jnp.pad(jnp.transpose(pred_confs[sl], (0, 2, 1)),
                          ((0, 0), (0, 0), (0, pad))).reshape(hb, NCLASSES,
                                                             RS, LS)
        outs.append(pl.pallas_call(
            _mbl_kernel,
            grid=(hb // IMGS,),
            in_specs=[
                pl.BlockSpec((IMGS, 4, RS, LS), lambda b: (b, 0, 0, 0)),
                pl.BlockSpec((IMGS, NCLASSES, RS, LS), lambda b: (b, 0, 0, 0)),
                pl.BlockSpec((IMGS, NOBJ, 6), lambda b: (b, 0, 0)),
                pl.BlockSpec((12, RS, LS), lambda b: (0, 0, 0)),
            ],
            out_specs=pl.BlockSpec((IMGS, 1, 8), lambda b: (b, 0, 0)),
            out_shape=jax.ShapeDtypeStruct((hb, 1, 8), jnp.float32),
            compiler_params=pltpu.CompilerParams(
                dimension_semantics=("parallel",),
            ),
        )(locs_p, confs_p, targets[sl], priors_p))
    out = jnp.concatenate(outs, axis=0)

    n_pos_total = jnp.sum(out[:, 0, 0])
    loc_total = jnp.sum(out[:, 0, 1])
    pos_ce_total = jnp.sum(out[:, 0, 2])
    hard_total = jnp.sum(out[:, 0, 3])
    conf_loss = (hard_total + pos_ce_total) / n_pos_total
    loc_loss = loc_total / (n_pos_total * 4.0)
    return (conf_loss, ALPHA * loc_loss)


# speedup vs baseline: 1.0969x; 1.0969x over previous
import jax
import jax.numpy as jnp
from jax.experimental import pallas as pl
from jax.experimental.pallas import tpu as pltpu

NBATCH = 64
NPRIORS = 8732
NCLASSES = 21
NOBJ = 20
OVERLAP_THRESHOLD = 0.5
NEGPOS_RATIO = 3
ALPHA = 1.0

RS = 72
LS = 128
PPAD = RS * LS

SEARCH_WAYS = 8
SEARCH_STEPS = 8
IMGS = 2


class _Fold:
    def __init__(self, op):
        self.op = op
        self.stack = []

    def push(self, v):
        r = 0
        while self.stack and self.stack[-1][0] == r:
            pr, pv = self.stack.pop()
            v = self.op(pv, v)
            r = pr + 1
        self.stack.append((r, v))

    def result(self):
        acc = self.stack[0][1]
        for _, v in self.stack[1:]:
            acc = self.op(acc, v)
        return acc


def _mbl_kernel(locs_ref, confs_ref, tgt_ref, priors_ref, out_ref):
    for i in range(IMGS):
        _one_image(locs_ref, confs_ref, tgt_ref, priors_ref, out_ref, i)


def _one_image(locs_ref, confs_ref, tgt_ref, priors_ref, out_ref, img):
    f32 = jnp.float32
    row_iota = jax.lax.broadcasted_iota(jnp.int32, (RS, LS), 0)
    lane_iota = jax.lax.broadcasted_iota(jnp.int32, (RS, LS), 1)
    idx = row_iota * LS + lane_iota
    valid = idx < NPRIORS

    px1 = priors_ref[0]
    py1 = priors_ref[1]
    px2 = priors_ref[2]
    py2 = priors_ref[3]
    area_b = priors_ref[4]
    pcx = priors_ref[5]
    pcy = priors_ref[6]
    rw10 = priors_ref[7]
    rh10 = priors_ref[8]
    rw = priors_ref[9]
    rh = priors_ref[10]

    def argmax_combine(a, b):
        va, ja = a
        vb, jb = b
        take = vb > va
        return jnp.maximum(va, vb), jnp.where(take, jb, ja)

    bps = []
    am = _Fold(argmax_combine)
    for j in range(NOBJ):
        ax1 = tgt_ref[img, j, 0]
        ay1 = tgt_ref[img, j, 1]
        ax2 = tgt_ref[img, j, 2]
        ay2 = tgt_ref[img, j, 3]
        area_a = (ax2 - ax1) * (ay2 - ay1)
        wx = jnp.maximum(jnp.minimum(ax2, px2) - jnp.maximum(ax1, px1), 0.0)
        wy = jnp.maximum(jnp.minimum(ay2, py2) - jnp.maximum(ay1, py1), 0.0)
        inter = wx * wy
        ov = inter / (area_a + area_b - inter)
        cmax = jnp.max(ov, axis=0, keepdims=True)
        rmin = jnp.min(jnp.where(ov == cmax, idx, PPAD),
                       axis=0, keepdims=True)
        rm = jnp.max(cmax, axis=1, keepdims=True)
        bps.append(jnp.min(jnp.where(cmax == rm, rmin, PPAD),
                           axis=1, keepdims=True))
        am.push((ov, jnp.full((RS, LS), f32(j))))
    best_ov, best_idx = am.result()

    wf = _Fold(jnp.maximum)
    for j in range(NOBJ):
        wf.push(jnp.where(idx == bps[j], f32(j), -1.0))
    winner = wf.result()
    forced = winner >= 0
    best_idx = jnp.where(forced, winner, best_idx)
    best_ov = jnp.where(forced, 1.0, best_ov)

    lab = jnp.zeros((RS, LS), f32)
    mx1 = jnp.zeros((RS, LS), f32)
    my1 = jnp.zeros((RS, LS), f32)
    mx2 = jnp.zeros((RS, LS), f32)
    my2 = jnp.zeros((RS, LS), f32)
    for j in range(NOBJ):
        sel = best_idx == f32(j)
        lab = lab + jnp.where(sel, tgt_ref[img, j, 4], 0.0)
        mx1 = mx1 + jnp.where(sel, tgt_ref[img, j, 0], 0.0)
        my1 = my1 + jnp.where(sel, tgt_ref[img, j, 1], 0.0)
        mx2 = mx2 + jnp.where(sel, tgt_ref[img, j, 2], 0.0)
        my2 = my2 + jnp.where(sel, tgt_ref[img, j, 3], 0.0)

    pos = jnp.logical_and(best_ov >= OVERLAP_THRESHOLD, valid)
    lab = jnp.where(pos, lab, 0.0)
    n_pos = jnp.sum(jnp.where(pos, 1.0, 0.0), keepdims=True)

    cx = (mx1 + mx2) / 2
    cy = (my1 + my2) / 2
    w = mx2 - mx1
    h = my2 - my1
    g_cx = (cx - pcx) * rw10
    g_cy = (cy - pcy) * rh10
    g_w = jnp.log(w * rw + 1e-10) * 5
    g_h = jnp.log(h * rh + 1e-10) * 5

    loc_abs = (jnp.where(pos, jnp.abs(locs_ref[img, 0] - g_cx), 0.0)
               + jnp.where(pos, jnp.abs(locs_ref[img, 1] - g_cy), 0.0)
               + jnp.where(pos, jnp.abs(locs_ref[img, 2] - g_w), 0.0)
               + jnp.where(pos, jnp.abs(locs_ref[img, 3] - g_h), 0.0))
    loc_sum = jnp.sum(loc_abs, keepdims=True)

    sf = _Fold(jnp.add)
    for c in range(NCLASSES):
        sf.push(jnp.exp(confs_ref[img, c]))
    lse = jnp.log(sf.result())
    pf = _Fold(jnp.add)
    pf.push(jnp.where(lab == 0.0, confs_ref[img, 0], 0.0))
    for c in range(1, NCLASSES):
        pf.push(jnp.where(lab == f32(c), confs_ref[img, c], 0.0))
    ce = lse - pf.result()

    pos_ce_sum = jnp.sum(jnp.where(pos, ce, 0.0), keepdims=True)

    ce_neg = jnp.where(jnp.logical_or(pos, jnp.logical_not(valid)), 0.0, ce)
    k = jnp.minimum(n_pos * NEGPOS_RATIO, f32(NPRIORS))

    lo = jnp.full((1, 1), -1.0, f32)
    wdt = jnp.max(ce_neg, keepdims=True) + 2.0
    inv = 1.0 / SEARCH_WAYS
    for _ in range(SEARCH_STEPS):
        ws = wdt * inv
        cnts = [jnp.sum(jnp.where(ce_neg > lo + ws * i, 1.0, 0.0),
                        keepdims=True) for i in range(1, SEARCH_WAYS)]
        jf = _Fold(jnp.add)
        for c in cnts:
            jf.push(jnp.where(c > k, 1.0, 0.0))
        lo = lo + ws * jf.result()
        wdt = ws
    hi = lo + wdt
    above = ce_neg > hi
    cnt_hi = jnp.sum(jnp.where(above, 1.0, 0.0), keepdims=True)
    top_sum = jnp.sum(jnp.where(above, ce_neg, 0.0), keepdims=True)
    vnext = jnp.max(jnp.where(above, -1.0, ce_neg), keepdims=True)
    min_above = jnp.min(jnp.where(above, ce_neg, 3.4e38), keepdims=True)
    rem = k - cnt_hi
    hard_sum = top_sum + jnp.where(rem >= 0, rem * vnext, rem * min_above)

    lane8 = jax.lax.broadcasted_iota(jnp.int32, (1, 8), 1)
    row = (jnp.where(lane8 == 0, n_pos, 0.0)
           + jnp.where(lane8 == 1, loc_sum, 0.0)
           + jnp.where(lane8 == 2, pos_ce_sum, 0.0)
           + jnp.where(lane8 == 3, hard_sum, 0.0))
    out_ref[img] = row


NSPLIT = 1


def kernel(pred_locs, pred_confs, targets, priors):
    pad = PPAD - NPRIORS
    hb = NBATCH // NSPLIT

    pcx, pcy, pw, ph = priors[:, 0], priors[:, 1], priors[:, 2], priors[:, 3]
    px1 = pcx - pw / 2
    py1 = pcy - ph / 2
    px2 = pcx + pw / 2
    py2 = pcy + ph / 2
    area_b = (px2 - px1) * (py2 - py1)
    chans = jnp.stack([px1, py1, px2, py2, area_b, pcx, pcy,
                       10.0 / pw, 10.0 / ph, 1.0 / pw, 1.0 / ph,
                       jnp.zeros_like(pw)])
    priors_p = jnp.pad(chans, ((0, 0), (0, pad))).reshape(12, RS, LS)

    outs = []
    for h in range(NSPLIT):
        sl = slice(h * hb, (h + 1) * hb)
        locs_p = jnp.pad(jnp.transpose(pred_locs[sl], (0, 2, 1)),
                         ((0, 0), (0, 0), (0, pad))).reshape(hb, 4, RS, LS)
        confs_p = jnp.pad(jnp.transpose(pred_confs[sl], (0, 2, 1)),
                          ((0, 0), (0, 0), (0, pad))).reshape(hb, NCLASSES,
                                                             RS, LS)
        outs.append(pl.pallas_call(
            _mbl_kernel,
            grid=(hb // IMGS,),
            in_specs=[
                pl.BlockSpec((IMGS, 4, RS, LS), lambda b: (b, 0, 0, 0)),
                pl.BlockSpec((IMGS, NCLASSES, RS, LS), lambda b: (b, 0, 0, 0)),
                pl.BlockSpec((IMGS, NOBJ, 6), lambda b: (b, 0, 0)),
                pl.BlockSpec((12, RS, LS), lambda b: (0, 0, 0)),
            ],
            out_specs=pl.BlockSpec((IMGS, 1, 8), lambda b: (b, 0, 0)),
            out_shape=jax.ShapeDtypeStruct((hb, 1, 8), jnp.float32),
            compiler_params=pltpu.CompilerParams(
                dimension_semantics=("parallel",),
            ),
        )(locs_p, confs_p, targets[sl], priors_p))
    out = jnp.concatenate(outs, axis=0)

    n_pos_total = jnp.sum(out[:, 0, 0])
    loc_total = jnp.sum(out[:, 0, 1])
    pos_ce_total = jnp.sum(out[:, 0, 2])
    hard_total = jnp.sum(out[:, 0, 3])
    conf_loss = (hard_total + pos_ce_total) / n_pos_total
    loc_loss = loc_total / (n_pos_total * 4.0)
    return (conf_loss, ALPHA * loc_loss)


# speedup vs baseline: 1.1019x; 1.0045x over previous
import jax
import jax.numpy as jnp
from jax.experimental import pallas as pl
from jax.experimental.pallas import tpu as pltpu

NBATCH = 64
NPRIORS = 8732
NCLASSES = 21
NOBJ = 20
OVERLAP_THRESHOLD = 0.5
NEGPOS_RATIO = 3
ALPHA = 1.0

RS = 72
LS = 128
PPAD = RS * LS

SEARCH_WAYS = 8
SEARCH_STEPS = 8
IMGS = 4


class _Fold:
    def __init__(self, op):
        self.op = op
        self.stack = []

    def push(self, v):
        r = 0
        while self.stack and self.stack[-1][0] == r:
            pr, pv = self.stack.pop()
            v = self.op(pv, v)
            r = pr + 1
        self.stack.append((r, v))

    def result(self):
        acc = self.stack[0][1]
        for _, v in self.stack[1:]:
            acc = self.op(acc, v)
        return acc


def _mbl_kernel(locs_ref, confs_ref, tgt_ref, priors_ref, out_ref):
    for i in range(IMGS):
        _one_image(locs_ref, confs_ref, tgt_ref, priors_ref, out_ref, i)


def _one_image(locs_ref, confs_ref, tgt_ref, priors_ref, out_ref, img):
    f32 = jnp.float32
    row_iota = jax.lax.broadcasted_iota(jnp.int32, (RS, LS), 0)
    lane_iota = jax.lax.broadcasted_iota(jnp.int32, (RS, LS), 1)
    idx = row_iota * LS + lane_iota
    valid = idx < NPRIORS

    px1 = priors_ref[0]
    py1 = priors_ref[1]
    px2 = priors_ref[2]
    py2 = priors_ref[3]
    area_b = priors_ref[4]
    pcx = priors_ref[5]
    pcy = priors_ref[6]
    rw10 = priors_ref[7]
    rh10 = priors_ref[8]
    rw = priors_ref[9]
    rh = priors_ref[10]

    def argmax_combine(a, b):
        va, ja = a
        vb, jb = b
        take = vb > va
        return jnp.maximum(va, vb), jnp.where(take, jb, ja)

    bps = []
    am = _Fold(argmax_combine)
    for j in range(NOBJ):
        ax1 = tgt_ref[img, j, 0]
        ay1 = tgt_ref[img, j, 1]
        ax2 = tgt_ref[img, j, 2]
        ay2 = tgt_ref[img, j, 3]
        area_a = (ax2 - ax1) * (ay2 - ay1)
        wx = jnp.maximum(jnp.minimum(ax2, px2) - jnp.maximum(ax1, px1), 0.0)
        wy = jnp.maximum(jnp.minimum(ay2, py2) - jnp.maximum(ay1, py1), 0.0)
        inter = wx * wy
        ov = inter / (area_a + area_b - inter)
        cmax = jnp.max(ov, axis=0, keepdims=True)
        rmin = jnp.min(jnp.where(ov == cmax, idx, PPAD),
                       axis=0, keepdims=True)
        rm = jnp.max(cmax, axis=1, keepdims=True)
        bps.append(jnp.min(jnp.where(cmax == rm, rmin, PPAD),
                           axis=1, keepdims=True))
        am.push((ov, jnp.full((RS, LS), f32(j))))
    best_ov, best_idx = am.result()

    wf = _Fold(jnp.maximum)
    for j in range(NOBJ):
        wf.push(jnp.where(idx == bps[j], f32(j), -1.0))
    winner = wf.result()
    forced = winner >= 0
    best_idx = jnp.where(forced, winner, best_idx)
    best_ov = jnp.where(forced, 1.0, best_ov)

    lab = jnp.zeros((RS, LS), f32)
    mx1 = jnp.zeros((RS, LS), f32)
    my1 = jnp.zeros((RS, LS), f32)
    mx2 = jnp.zeros((RS, LS), f32)
    my2 = jnp.zeros((RS, LS), f32)
    for j in range(NOBJ):
        sel = best_idx == f32(j)
        lab = lab + jnp.where(sel, tgt_ref[img, j, 4], 0.0)
        mx1 = mx1 + jnp.where(sel, tgt_ref[img, j, 0], 0.0)
        my1 = my1 + jnp.where(sel, tgt_ref[img, j, 1], 0.0)
        mx2 = mx2 + jnp.where(sel, tgt_ref[img, j, 2], 0.0)
        my2 = my2 + jnp.where(sel, tgt_ref[img, j, 3], 0.0)

    pos = jnp.logical_and(best_ov >= OVERLAP_THRESHOLD, valid)
    lab = jnp.where(pos, lab, 0.0)
    n_pos = jnp.sum(jnp.where(pos, 1.0, 0.0), keepdims=True)

    cx = (mx1 + mx2) / 2
    cy = (my1 + my2) / 2
    w = mx2 - mx1
    h = my2 - my1
    g_cx = (cx - pcx) * rw10
    g_cy = (cy - pcy) * rh10
    g_w = jnp.log(w * rw + 1e-10) * 5
    g_h = jnp.log(h * rh + 1e-10) * 5

    loc_abs = (jnp.where(pos, jnp.abs(locs_ref[img, 0] - g_cx), 0.0)
               + jnp.where(pos, jnp.abs(locs_ref[img, 1] - g_cy), 0.0)
               + jnp.where(pos, jnp.abs(locs_ref[img, 2] - g_w), 0.0)
               + jnp.where(pos, jnp.abs(locs_ref[img, 3] - g_h), 0.0))
    loc_sum = jnp.sum(loc_abs, keepdims=True)

    sf = _Fold(jnp.add)
    for c in range(NCLASSES):
        sf.push(jnp.exp(confs_ref[img, c]))
    lse = jnp.log(sf.result())
    pf = _Fold(jnp.add)
    pf.push(jnp.where(lab == 0.0, confs_ref[img, 0], 0.0))
    for c in range(1, NCLASSES):
        pf.push(jnp.where(lab == f32(c), confs_ref[img, c], 0.0))
    ce = lse - pf.result()

    pos_ce_sum = jnp.sum(jnp.where(pos, ce, 0.0), keepdims=True)

    ce_neg = jnp.where(jnp.logical_or(pos, jnp.logical_not(valid)), 0.0, ce)
    k = jnp.minimum(n_pos * NEGPOS_RATIO, f32(NPRIORS))

    lo = jnp.full((1, 1), -1.0, f32)
    wdt = jnp.max(ce_neg, keepdims=True) + 2.0
    inv = 1.0 / SEARCH_WAYS
    for _ in range(SEARCH_STEPS):
        ws = wdt * inv
        cnts = [jnp.sum(jnp.where(ce_neg > lo + ws * i, 1.0, 0.0),
                        keepdims=True) for i in range(1, SEARCH_WAYS)]
        jf = _Fold(jnp.add)
        for c in cnts:
            jf.push(jnp.where(c > k, 1.0, 0.0))
        lo = lo + ws * jf.result()
        wdt = ws
    hi = lo + wdt
    above = ce_neg > hi
    cnt_hi = jnp.sum(jnp.where(above, 1.0, 0.0), keepdims=True)
    top_sum = jnp.sum(jnp.where(above, ce_neg, 0.0), keepdims=True)
    vnext = jnp.max(jnp.where(above, -1.0, ce_neg), keepdims=True)
    min_above = jnp.min(jnp.where(above, ce_neg, 3.4e38), keepdims=True)
    rem = k - cnt_hi
    hard_sum = top_sum + jnp.where(rem >= 0, rem * vnext, rem * min_above)

    lane8 = jax.lax.broadcasted_iota(jnp.int32, (1, 8), 1)
    row = (jnp.where(lane8 == 0, n_pos, 0.0)
           + jnp.where(lane8 == 1, loc_sum, 0.0)
           + jnp.where(lane8 == 2, pos_ce_sum, 0.0)
           + jnp.where(lane8 == 3, hard_sum, 0.0))
    out_ref[img] = row


NSPLIT = 1


def kernel(pred_locs, pred_confs, targets, priors):
    pad = PPAD - NPRIORS
    hb = NBATCH // NSPLIT

    pcx, pcy, pw, ph = priors[:, 0], priors[:, 1], priors[:, 2], priors[:, 3]
    px1 = pcx - pw / 2
    py1 = pcy - ph / 2
    px2 = pcx + pw / 2
    py2 = pcy + ph / 2
    area_b = (px2 - px1) * (py2 - py1)
    chans = jnp.stack([px1, py1, px2, py2, area_b, pcx, pcy,
                       10.0 / pw, 10.0 / ph, 1.0 / pw, 1.0 / ph,
                       jnp.zeros_like(pw)])
    priors_p = jnp.pad(chans, ((0, 0), (0, pad))).reshape(12, RS, LS)

    outs = []
    for h in range(NSPLIT):
        sl = slice(h * hb, (h + 1) * hb)
        locs_p = jnp.pad(jnp.transpose(pred_locs[sl], (0, 2, 1)),
                         ((0, 0), (0, 0), (0, pad))).reshape(hb, 4, RS, LS)
        confs_p = jnp.pad(jnp.transpose(pred_confs[sl], (0, 2, 1)),
                          ((0, 0), (0, 0), (0, pad))).reshape(hb, NCLASSES,
                                                             RS, LS)
        outs.append(pl.pallas_call(
            _mbl_kernel,
            grid=(hb // IMGS,),
            in_specs=[
                pl.BlockSpec((IMGS, 4, RS, LS), lambda b: (b, 0, 0, 0)),
                pl.BlockSpec((IMGS, NCLASSES, RS, LS), lambda b: (b, 0, 0, 0)),
                pl.BlockSpec((IMGS, NOBJ, 6), lambda b: (b, 0, 0)),
                pl.BlockSpec((12, RS, LS), lambda b: (0, 0, 0)),
            ],
            out_specs=pl.BlockSpec((IMGS, 1, 8), lambda b: (b, 0, 0)),
            out_shape=jax.ShapeDtypeStruct((hb, 1, 8), jnp.float32),
            compiler_params=pltpu.CompilerParams(
                dimension_semantics=("parallel",),
            ),
        )(locs_p, confs_p, targets[sl], priors_p))
    out = jnp.concatenate(outs, axis=0)

    n_pos_total = jnp.sum(out[:, 0, 0])
    loc_total = jnp.sum(out[:, 0, 1])
    pos_ce_total = jnp.sum(out[:, 0, 2])
    hard_total = jnp.sum(out[:, 0, 3])
    conf_loss = (hard_total + pos_ce_total) / n_pos_total
    loc_loss = loc_total / (n_pos_total * 4.0)
    return (conf_loss, ALPHA * loc_loss)
